# baseline (speedup 1.0000x reference)
import jax
import jax.numpy as jnp
from jax.experimental import pallas as pl

HU = (-1000.0, -75.0, 0.0, 15.0, 25.0, 40.0, 50.0, 200.0, 1000.0)
NPTS = 9
P = 4
VPP = P * P * P
VD = 16
OUT = 768
BN = 512


def _u_kernel(vec_ref, wt_ref, u_ref):
    wt = wt_ref[...]
    for k in range(NPTS):
        vcol = vec_ref[:, k : k + 1]
        prod = vcol * wt
        uk = prod.reshape(VPP, VD, OUT).sum(axis=1)
        u_ref[k * VPP : (k + 1) * VPP, :] = uk.astype(jnp.bfloat16)


def _main_kernel(x_ref, u_ref, b_ref, o_ref):
    xb = x_ref[...].reshape(BN, VPP)
    hats = []
    inv = [1.0 / (HU[k + 1] - HU[k]) for k in range(NPTS - 1)]
    hats.append(jnp.clip((HU[1] - xb) * inv[0], 0.0, 1.0))
    for k in range(1, NPTS - 1):
        up = (xb - HU[k - 1]) * inv[k - 1]
        dn = (HU[k + 1] - xb) * inv[k]
        hats.append(jnp.maximum(jnp.minimum(up, dn), 0.0))
    hats.append(jnp.clip((xb - HU[NPTS - 2]) * inv[NPTS - 2], 0.0, 1.0))
    a = jnp.concatenate(hats, axis=1).astype(jnp.bfloat16)
    acc = jax.lax.dot_general(
        u_ref[...], a, (((0,), (1,)), ((), ())),
        preferred_element_type=jnp.float32,
    )
    o_ref[...] = (acc + b_ref[...]).reshape(1, OUT, BN)


def kernel(x, vectors, Wfc, bfc):
    b, c, d, h, w = x.shape
    nd, nh, nw = d // P, h // P, w // P
    npat = nd * nh * nw
    xp = x.reshape(b, nd, P, nh, P, nw, P)
    xp = xp.transpose(0, 1, 3, 5, 2, 4, 6).reshape(b, npat, VPP)

    vec_t = vectors.reshape(NPTS, VPP * VD).T
    wt = Wfc.T

    u = pl.pallas_call(
        _u_kernel,
        out_shape=jax.ShapeDtypeStruct((NPTS * VPP, OUT), jnp.bfloat16),
    )(vec_t, wt)

    grid = (b, npat // BN)
    out = pl.pallas_call(
        _main_kernel,
        grid=grid,
        in_specs=[
            pl.BlockSpec((1, BN, VPP), lambda i, j: (i, j, 0)),
            pl.BlockSpec((NPTS * VPP, OUT), lambda i, j: (0, 0)),
            pl.BlockSpec((OUT, 1), lambda i, j: (0, 0)),
        ],
        out_specs=pl.BlockSpec((1, OUT, BN), lambda i, j: (i, 0, j)),
        out_shape=jax.ShapeDtypeStruct((b, OUT, npat), jnp.float32),
    )(xp, u, bfc.reshape(OUT, 1))

    return out.reshape(b, OUT, nd, nh, nw)


# speedup vs baseline: 1.5325x; 1.5325x over previous
import jax
import jax.numpy as jnp
from jax.experimental import pallas as pl

HU = (-1000.0, -75.0, 0.0, 15.0, 25.0, 40.0, 50.0, 200.0, 1000.0)
NPTS = 9
P = 4
VPP = P * P * P
VD = 16
OUT = 768


def _u_kernel(vec_ref, wt_ref, u_ref):
    wt = wt_ref[...]
    for k in range(NPTS):
        vcol = vec_ref[:, k : k + 1]
        prod = vcol * wt
        uk = prod.reshape(VPP, VD, OUT).sum(axis=1)
        u_ref[k * VPP : (k + 1) * VPP, :] = uk.astype(jnp.bfloat16)


def _main_kernel(x_ref, u_ref, b_ref, o_ref):
    dsl = x_ref.shape[1]
    nh = x_ref.shape[3]
    nw = x_ref.shape[5]
    npat = dsl * nh * nw
    xs = x_ref[...].reshape(dsl, P, nh, P, nw, P)
    xt = jnp.transpose(xs, (0, 2, 4, 1, 3, 5)).reshape(npat, VPP)
    hats = []
    inv = [1.0 / (HU[k + 1] - HU[k]) for k in range(NPTS - 1)]
    hats.append(jnp.clip((HU[1] - xt) * inv[0], 0.0, 1.0))
    for k in range(1, NPTS - 1):
        up = (xt - HU[k - 1]) * inv[k - 1]
        dn = (HU[k + 1] - xt) * inv[k]
        hats.append(jnp.maximum(jnp.minimum(up, dn), 0.0))
    hats.append(jnp.clip((xt - HU[NPTS - 2]) * inv[NPTS - 2], 0.0, 1.0))
    a = jnp.concatenate(hats, axis=1).astype(jnp.bfloat16)
    acc = jax.lax.dot_general(
        u_ref[...], a, (((0,), (1,)), ((), ())),
        preferred_element_type=jnp.float32,
    )
    o_ref[...] = (acc + b_ref[...]).reshape(1, OUT, npat)


def kernel(x, vectors, Wfc, bfc):
    b, c, d, h, w = x.shape
    nd, nh, nw = d // P, h // P, w // P
    npat = nh * nw
    x8 = x.reshape(b, nd, P, nh, P, nw, P)

    vec_t = vectors.reshape(NPTS, VPP * VD).T
    wt = Wfc.T

    u = pl.pallas_call(
        _u_kernel,
        out_shape=jax.ShapeDtypeStruct((NPTS * VPP, OUT), jnp.bfloat16),
    )(vec_t, wt)

    dsl = 2
    grid = (b, nd // dsl)
    out = pl.pallas_call(
        _main_kernel,
        grid=grid,
        in_specs=[
            pl.BlockSpec((1, dsl, P, nh, P, nw, P),
                         lambda i, j: (i, j, 0, 0, 0, 0, 0)),
            pl.BlockSpec((NPTS * VPP, OUT), lambda i, j: (0, 0)),
            pl.BlockSpec((OUT, 1), lambda i, j: (0, 0)),
        ],
        out_specs=pl.BlockSpec((1, OUT, dsl * npat), lambda i, j: (i, 0, j)),
        out_shape=jax.ShapeDtypeStruct((b, OUT, nd * npat), jnp.float32),
    )(x8, u, bfc.reshape(OUT, 1))

    return out.reshape(b, OUT, nd, nh, nw)
